# baseline (speedup 1.0000x reference)
import functools

import jax
import jax.numpy as jnp
import numpy as np
from jax import lax
from jax.experimental import pallas as pl
from jax.experimental.pallas import tpu as pltpu
from jax.experimental.pallas import tpu_sc as plsc

_T = 2048
_H = 12
_C = 64
_NE = _H * _C
_M = 65536
_K = 3

_NEG = np.float32(-1e9)
_BIGF = np.float32(1e9)


def _qkv_body(x_ref, w_ref, o_ref):
    o_ref[...] = jnp.dot(x_ref[...], w_ref[...],
                         preferred_element_type=jnp.float32)


def _qkv_call(xf, w_attn):
    return pl.pallas_call(
        _qkv_body,
        grid=(3,),
        in_specs=[
            pl.BlockSpec((_T, _NE), lambda j: (0, 0)),
            pl.BlockSpec((_NE, _NE), lambda j: (0, j)),
        ],
        out_specs=pl.BlockSpec((_T, _NE), lambda j: (0, j)),
        out_shape=jax.ShapeDtypeStruct((_T, 3 * _NE), jnp.float32),
    )(xf, w_attn)


_QB = 512


def _attn_body(q_ref, k_ref, v_ref, o_ref):
    qb = pl.program_id(1)
    q2 = q_ref[...]
    k2 = k_ref[...]
    v2 = v_ref[...]
    rows = lax.broadcasted_iota(jnp.int32, (_QB, _T), 0) + qb * _QB
    cols = lax.broadcasted_iota(jnp.int32, (_QB, _T), 1)
    causal = cols <= rows
    halves = []
    for half in range(2):
        sl = slice(half * _C, (half + 1) * _C)
        s = lax.dot_general(q2[:, sl], k2[:, sl], (((1,), (1,)), ((), ())),
                            preferred_element_type=jnp.float32)
        s = s * np.float32(1.0 / (_C ** 0.5))
        s = jnp.where(causal, s, _NEG)
        a = jax.nn.softmax(s, axis=-1)
        halves.append(jnp.dot(a, v2[:, sl],
                              preferred_element_type=jnp.float32))
    o_ref[...] = jnp.concatenate(halves, axis=1)


def _attn_call(qkv):
    hp = _H // 2
    return pl.pallas_call(
        _attn_body,
        grid=(hp, _T // _QB),
        in_specs=[
            pl.BlockSpec((_QB, 2 * _C), lambda h2, qb: (qb, h2)),
            pl.BlockSpec((_T, 2 * _C), lambda h2, qb: (0, hp + h2)),
            pl.BlockSpec((_T, 2 * _C), lambda h2, qb: (0, 2 * hp + h2)),
        ],
        out_specs=pl.BlockSpec((_QB, 2 * _C), lambda h2, qb: (qb, h2)),
        out_shape=jax.ShapeDtypeStruct((_T, _NE), jnp.float32),
    )(qkv, qkv, qkv)


_MT = 1024


def _search_body(q_ref, mem_ref, o_ref, vals_ref, idxs_ref, mk_buf, sem):
    m = pl.program_id(0)
    nm = pl.num_programs(0)

    def key_dma(i, slot):
        return pltpu.make_async_copy(
            mem_ref.at[0, pl.ds(i * _MT, _MT), 0, :],
            mk_buf.at[slot], sem.at[slot])

    @pl.when(m == 0)
    def _():
        vals_ref[...] = jnp.full((_T, 8), _NEG, jnp.float32)
        idxs_ref[...] = jnp.full((_T, 8), _BIGF, jnp.float32)
        key_dma(0, 0).start()

    @pl.when(m + 1 < nm)
    def _():
        key_dma(m + 1, (m + 1) % 2).start()

    key_dma(m, m % 2).wait()
    q = q_ref[...]
    mk = mk_buf[m % 2]
    s = lax.dot_general(q, mk, (((1,), (1,)), ((), ())),
                        preferred_element_type=jnp.float32)

    rowid = lax.broadcasted_iota(
        jnp.int32, (_MT, 8), 0).astype(jnp.float32)
    lane0 = lax.broadcasted_iota(jnp.int32, (_MT, 8), 1) == 0
    colmat = jnp.where(lane0, rowid, jnp.float32(0.0))
    mf = (m * _MT).astype(jnp.float32)
    work = s
    bv, bi = [], []
    for j in range(_K):
        mj = jnp.max(work, axis=1, keepdims=True)
        is_m = work == mj
        onz = jnp.where(is_m, jnp.float32(1.0), jnp.float32(0.0))
        aj = jnp.dot(onz, colmat, precision=lax.Precision.HIGHEST,
                     preferred_element_type=jnp.float32)[:, 0:1] + mf
        bv.append(mj)
        bi.append(aj)
        if j < _K - 1:
            work = jnp.where(is_m, _NEG, work)

    av = [vals_ref[:, j:j + 1] for j in range(_K)]
    ai = [idxs_ref[:, j:j + 1] for j in range(_K)]
    pick = jnp.where
    c1 = bv[0] > av[0]
    r1v = pick(c1, bv[0], av[0])
    r1i = pick(c1, bi[0], ai[0])
    a2v = pick(c1, av[0], av[1])
    a2i = pick(c1, ai[0], ai[1])
    b2v = pick(c1, bv[1], bv[0])
    b2i = pick(c1, bi[1], bi[0])
    c2 = b2v > a2v
    r2v = pick(c2, b2v, a2v)
    r2i = pick(c2, b2i, a2i)
    a3v = pick(c2, a2v, pick(c1, av[1], av[2]))
    a3i = pick(c2, a2i, pick(c1, ai[1], ai[2]))
    b3v = pick(c2, pick(c1, bv[2], bv[1]), b2v)
    b3i = pick(c2, pick(c1, bi[2], bi[1]), b2i)
    c3 = b3v > a3v
    r3v = pick(c3, b3v, a3v)
    r3i = pick(c3, b3i, a3i)

    vals_ref[:, 0:1] = r1v
    vals_ref[:, 1:2] = r2v
    vals_ref[:, 2:3] = r3v
    idxs_ref[:, 0:1] = r1i
    idxs_ref[:, 1:2] = r2i
    idxs_ref[:, 2:3] = r3i

    @pl.when(m == nm - 1)
    def _():
        o_ref[...] = jnp.concatenate(
            [r1i, r2i, r3i, jnp.zeros((_T, 5), jnp.float32)],
            axis=1).astype(jnp.int32)


def _search_call(qkv, mem_kv):
    return pl.pallas_call(
        _search_body,
        grid=(_M // _MT,),
        in_specs=[
            pl.BlockSpec((_T, _NE), lambda m: (0, 0)),
            pl.BlockSpec(memory_space=pl.ANY),
        ],
        out_specs=pl.BlockSpec((_T, 8), lambda m: (0, 0)),
        out_shape=jax.ShapeDtypeStruct((_T, 8), jnp.int32),
        scratch_shapes=[
            pltpu.VMEM((_T, 8), jnp.float32),
            pltpu.VMEM((_T, 8), jnp.float32),
            pltpu.VMEM((2, _MT, _NE), jnp.float32),
            pltpu.SemaphoreType.DMA((2,)),
        ],
    )(qkv, mem_kv)


_NW = 32
_ROWS = _T * _K
_RPW = _ROWS // _NW
_CH = 32
_D = 2 * _NE


def _sc_gather_body(table_hbm, idx_hbm, out_hbm, idx_v, rows_v, sem):
    wid = lax.axis_index("s") * 2 + lax.axis_index("c")
    base = wid * _RPW
    for chunk in range(_RPW // _CH):
        off = base + chunk * _CH
        pltpu.sync_copy(idx_hbm.at[pl.ds(off, _CH)], idx_v)
        pltpu.async_copy(table_hbm.at[idx_v], rows_v, sem).wait()
        pltpu.sync_copy(rows_v, out_hbm.at[pl.ds(off, _CH)])


@functools.cache
def _sc_gather_kernel():
    return pl.kernel(
        _sc_gather_body,
        out_type=jax.ShapeDtypeStruct((_ROWS, 2, _NE), jnp.float32),
        mesh=plsc.VectorSubcoreMesh(core_axis_name="c",
                                    subcore_axis_name="s"),
        scratch_types=[
            pltpu.VMEM((_CH,), jnp.int32),
            pltpu.VMEM((_CH, 2, _NE), jnp.float32),
            pltpu.SemaphoreType.DMA,
        ],
    )


def _gather_call(mem3, idx):
    return _sc_gather_kernel()(mem3, idx)


_SCALE = np.float32(_NE / (_H ** (-0.5)))


def _final_body(q_ref, y_ref, mk0, mk1, mk2, mv0, mv1, mv2,
                g_ref, wp_ref, o_ref):
    h2 = pl.program_id(0)
    q2 = q_ref[...]
    qk0, qk1 = [], []
    for mk in (mk0, mk1, mk2):
        p = q2 * mk[...]
        qk0.append(jnp.sum(p[:, :_C], axis=1, keepdims=True) * _SCALE)
        qk1.append(jnp.sum(p[:, _C:], axis=1, keepdims=True) * _SCALE)
    w0 = jax.nn.softmax(jnp.concatenate(qk0, axis=1), axis=1)
    w1 = jax.nn.softmax(jnp.concatenate(qk1, axis=1), axis=1)
    mem0 = jnp.zeros((_T, _C), jnp.float32)
    mem1 = jnp.zeros((_T, _C), jnp.float32)
    for k, mv in enumerate((mv0, mv1, mv2)):
        mvv = mv[...]
        mem0 += w0[:, k:k + 1] * mvv[:, :_C]
        mem1 += w1[:, k:k + 1] * mvv[:, _C:]
    mem = jnp.concatenate([mem0, mem1], axis=1)
    g = g_ref[...]
    comb = mem * g + y_ref[...] * (np.float32(1.0) - g)
    acc = jnp.dot(comb, wp_ref[...], preferred_element_type=jnp.float32)

    @pl.when(h2 == 0)
    def _():
        o_ref[...] = jnp.zeros_like(o_ref)

    o_ref[...] += acc


def _final_call(qkv, y, sel_r, g_row, w_proj):
    hp = _H // 2
    kpart = [pl.BlockSpec((_T, 2 * _C), (lambda h2, k=k: (0, k * 12 + h2)))
             for k in range(_K)]
    vpart = [pl.BlockSpec((_T, 2 * _C), (lambda h2, k=k: (0, k * 12 + hp + h2)))
             for k in range(_K)]
    return pl.pallas_call(
        _final_body,
        grid=(hp,),
        in_specs=[
            pl.BlockSpec((_T, 2 * _C), lambda h2: (0, h2)),
            pl.BlockSpec((_T, 2 * _C), lambda h2: (0, h2)),
            *kpart, *vpart,
            pl.BlockSpec((1, 2 * _C), lambda h2: (0, h2)),
            pl.BlockSpec((2 * _C, _NE), lambda h2: (h2, 0)),
        ],
        out_specs=pl.BlockSpec((_T, _NE), lambda h2: (0, 0)),
        out_shape=jax.ShapeDtypeStruct((_T, _NE), jnp.float32),
    )(qkv, y, sel_r, sel_r, sel_r, sel_r, sel_r, sel_r, g_row, w_proj)


def kernel(x, mem_kv, W_attn, W_proj, gate_bias):
    b, t, h, c = x.shape
    xf = x.reshape(t, h * c)
    mem3 = mem_kv.reshape(_M, 2, _NE)

    qkv = _qkv_call(xf, W_attn)
    y = _attn_call(qkv)
    topi = _search_call(qkv, mem_kv)
    idx = jnp.clip(topi[:, :_K], 0, _M - 1).reshape(_ROWS)
    sel = _gather_call(mem3, idx)
    sel_r = sel.reshape(_T, _K * _D)
    g_row = jnp.repeat(gate_bias.reshape(_H), _C).reshape(1, _NE)
    out = _final_call(qkv, y, sel_r, g_row, W_proj)
    return out.reshape(b, t, h * c)


# speedup vs baseline: 2.3746x; 2.3746x over previous
import functools

import jax
import jax.numpy as jnp
import numpy as np
from jax import lax
from jax.experimental import pallas as pl
from jax.experimental.pallas import tpu as pltpu
from jax.experimental.pallas import tpu_sc as plsc

_T = 2048
_H = 12
_C = 64
_NE = _H * _C
_M = 65536
_K = 3

_NEG = np.float32(-1e9)
_BIGF = np.float32(1e9)


def _qkv_body(x_ref, w_ref, o_ref):
    o_ref[...] = jnp.dot(x_ref[...], w_ref[...],
                         preferred_element_type=jnp.float32)


def _qkv_call(xf, w_attn):
    return pl.pallas_call(
        _qkv_body,
        grid=(3,),
        in_specs=[
            pl.BlockSpec((_T, _NE), lambda j: (0, 0)),
            pl.BlockSpec((_NE, _NE), lambda j: (0, j)),
        ],
        out_specs=pl.BlockSpec((_T, _NE), lambda j: (0, j)),
        out_shape=jax.ShapeDtypeStruct((_T, 3 * _NE), jnp.float32),
    )(xf, w_attn)


_QB = 512


def _attn_body(q_ref, k_ref, v_ref, o_ref):
    qb = pl.program_id(1)
    q2 = q_ref[...]
    k2 = k_ref[...]
    v2 = v_ref[...]
    rows = lax.broadcasted_iota(jnp.int32, (_QB, _T), 0) + qb * _QB
    cols = lax.broadcasted_iota(jnp.int32, (_QB, _T), 1)
    causal = cols <= rows
    halves = []
    for half in range(2):
        sl = slice(half * _C, (half + 1) * _C)
        s = lax.dot_general(q2[:, sl], k2[:, sl], (((1,), (1,)), ((), ())),
                            preferred_element_type=jnp.float32)
        s = s * np.float32(1.0 / (_C ** 0.5))
        s = jnp.where(causal, s, _NEG)
        a = jax.nn.softmax(s, axis=-1)
        halves.append(jnp.dot(a, v2[:, sl],
                              preferred_element_type=jnp.float32))
    o_ref[...] = jnp.concatenate(halves, axis=1)


def _attn_call(qkv):
    hp = _H // 2
    return pl.pallas_call(
        _attn_body,
        grid=(hp, _T // _QB),
        in_specs=[
            pl.BlockSpec((_QB, 2 * _C), lambda h2, qb: (qb, h2)),
            pl.BlockSpec((_T, 2 * _C), lambda h2, qb: (0, hp + h2)),
            pl.BlockSpec((_T, 2 * _C), lambda h2, qb: (0, 2 * hp + h2)),
        ],
        out_specs=pl.BlockSpec((_QB, 2 * _C), lambda h2, qb: (qb, h2)),
        out_shape=jax.ShapeDtypeStruct((_T, _NE), jnp.float32),
    )(qkv, qkv, qkv)


_MT = 2048


def _search_body(q_ref, mem_ref, o_ref, vals_ref, idxs_ref, mk_buf, sem):
    m = pl.program_id(0)
    nm = pl.num_programs(0)

    def key_dma(i, slot):
        return pltpu.make_async_copy(
            mem_ref.at[0, pl.ds(i * _MT, _MT), 0, :],
            mk_buf.at[slot], sem.at[slot])

    @pl.when(m == 0)
    def _():
        vals_ref[...] = jnp.full((_T, 8), _NEG, jnp.float32)
        idxs_ref[...] = jnp.full((_T, 8), _BIGF, jnp.float32)
        key_dma(0, 0).start()

    @pl.when(m + 1 < nm)
    def _():
        key_dma(m + 1, (m + 1) % 2).start()

    key_dma(m, m % 2).wait()
    q = q_ref[...]
    mk = mk_buf[m % 2]
    s = lax.dot_general(q, mk, (((1,), (1,)), ((), ())),
                        preferred_element_type=jnp.float32)

    colrow = lax.broadcasted_iota(
        jnp.int32, (1, _MT), 1).astype(jnp.float32)
    mf = (m * _MT).astype(jnp.float32)
    work = s
    bv, bi = [], []
    for j in range(_K):
        mj = jnp.max(work, axis=1, keepdims=True)
        is_m = work == mj
        aj = jnp.min(jnp.where(is_m, colrow, _BIGF),
                     axis=1, keepdims=True) + mf
        bv.append(mj)
        bi.append(aj)
        if j < _K - 1:
            work = jnp.where(is_m, _NEG, work)

    av = [vals_ref[:, j:j + 1] for j in range(_K)]
    ai = [idxs_ref[:, j:j + 1] for j in range(_K)]
    pick = jnp.where
    c1 = bv[0] > av[0]
    r1v = pick(c1, bv[0], av[0])
    r1i = pick(c1, bi[0], ai[0])
    a2v = pick(c1, av[0], av[1])
    a2i = pick(c1, ai[0], ai[1])
    b2v = pick(c1, bv[1], bv[0])
    b2i = pick(c1, bi[1], bi[0])
    c2 = b2v > a2v
    r2v = pick(c2, b2v, a2v)
    r2i = pick(c2, b2i, a2i)
    a3v = pick(c2, a2v, pick(c1, av[1], av[2]))
    a3i = pick(c2, a2i, pick(c1, ai[1], ai[2]))
    b3v = pick(c2, pick(c1, bv[2], bv[1]), b2v)
    b3i = pick(c2, pick(c1, bi[2], bi[1]), b2i)
    c3 = b3v > a3v
    r3v = pick(c3, b3v, a3v)
    r3i = pick(c3, b3i, a3i)

    vals_ref[:, 0:1] = r1v
    vals_ref[:, 1:2] = r2v
    vals_ref[:, 2:3] = r3v
    idxs_ref[:, 0:1] = r1i
    idxs_ref[:, 1:2] = r2i
    idxs_ref[:, 2:3] = r3i

    @pl.when(m == nm - 1)
    def _():
        o_ref[...] = jnp.concatenate(
            [r1i, r2i, r3i, jnp.zeros((_T, 5), jnp.float32)],
            axis=1).astype(jnp.int32)


def _search_call(qkv, mem_kv):
    return pl.pallas_call(
        _search_body,
        grid=(_M // _MT,),
        in_specs=[
            pl.BlockSpec((_T, _NE), lambda m: (0, 0)),
            pl.BlockSpec(memory_space=pl.ANY),
        ],
        out_specs=pl.BlockSpec((_T, 8), lambda m: (0, 0)),
        out_shape=jax.ShapeDtypeStruct((_T, 8), jnp.int32),
        scratch_shapes=[
            pltpu.VMEM((_T, 8), jnp.float32),
            pltpu.VMEM((_T, 8), jnp.float32),
            pltpu.VMEM((2, _MT, _NE), jnp.float32),
            pltpu.SemaphoreType.DMA((2,)),
        ],
    )(qkv, mem_kv)


_NW = 32
_ROWS = _T * _K
_RPW = _ROWS // _NW
_CH = 32
_D = 2 * _NE


def _sc_gather_body(table_hbm, idx_hbm, out_hbm, idx_v, rows_v, sem):
    wid = lax.axis_index("s") * 2 + lax.axis_index("c")
    base = wid * _RPW
    for chunk in range(_RPW // _CH):
        off = base + chunk * _CH
        pltpu.sync_copy(idx_hbm.at[pl.ds(off, _CH)], idx_v)
        pltpu.async_copy(table_hbm.at[idx_v], rows_v, sem).wait()
        pltpu.sync_copy(rows_v, out_hbm.at[pl.ds(off, _CH)])


@functools.cache
def _sc_gather_kernel():
    return pl.kernel(
        _sc_gather_body,
        out_type=jax.ShapeDtypeStruct((_ROWS, 2, _NE), jnp.float32),
        mesh=plsc.VectorSubcoreMesh(core_axis_name="c",
                                    subcore_axis_name="s"),
        scratch_types=[
            pltpu.VMEM((_CH,), jnp.int32),
            pltpu.VMEM((_CH, 2, _NE), jnp.float32),
            pltpu.SemaphoreType.DMA,
        ],
    )


def _gather_call(mem3, idx):
    return _sc_gather_kernel()(mem3, idx)


_SCALE = np.float32(_NE / (_H ** (-0.5)))


def _final_body(q_ref, y_ref, mk0, mk1, mk2, mv0, mv1, mv2,
                g_ref, wp_ref, o_ref):
    h2 = pl.program_id(0)
    q2 = q_ref[...]
    qk0, qk1 = [], []
    for mk in (mk0, mk1, mk2):
        p = q2 * mk[...]
        qk0.append(jnp.sum(p[:, :_C], axis=1, keepdims=True) * _SCALE)
        qk1.append(jnp.sum(p[:, _C:], axis=1, keepdims=True) * _SCALE)
    w0 = jax.nn.softmax(jnp.concatenate(qk0, axis=1), axis=1)
    w1 = jax.nn.softmax(jnp.concatenate(qk1, axis=1), axis=1)
    mem0 = jnp.zeros((_T, _C), jnp.float32)
    mem1 = jnp.zeros((_T, _C), jnp.float32)
    for k, mv in enumerate((mv0, mv1, mv2)):
        mvv = mv[...]
        mem0 += w0[:, k:k + 1] * mvv[:, :_C]
        mem1 += w1[:, k:k + 1] * mvv[:, _C:]
    mem = jnp.concatenate([mem0, mem1], axis=1)
    g = g_ref[...]
    comb = mem * g + y_ref[...] * (np.float32(1.0) - g)
    acc = jnp.dot(comb, wp_ref[...], preferred_element_type=jnp.float32)

    @pl.when(h2 == 0)
    def _():
        o_ref[...] = jnp.zeros_like(o_ref)

    o_ref[...] += acc


def _final_call(qkv, y, sel_r, g_row, w_proj):
    hp = _H // 2
    kpart = [pl.BlockSpec((_T, 2 * _C), (lambda h2, k=k: (0, k * 12 + h2)))
             for k in range(_K)]
    vpart = [pl.BlockSpec((_T, 2 * _C), (lambda h2, k=k: (0, k * 12 + hp + h2)))
             for k in range(_K)]
    return pl.pallas_call(
        _final_body,
        grid=(hp,),
        in_specs=[
            pl.BlockSpec((_T, 2 * _C), lambda h2: (0, h2)),
            pl.BlockSpec((_T, 2 * _C), lambda h2: (0, h2)),
            *kpart, *vpart,
            pl.BlockSpec((1, 2 * _C), lambda h2: (0, h2)),
            pl.BlockSpec((2 * _C, _NE), lambda h2: (h2, 0)),
        ],
        out_specs=pl.BlockSpec((_T, _NE), lambda h2: (0, 0)),
        out_shape=jax.ShapeDtypeStruct((_T, _NE), jnp.float32),
    )(qkv, y, sel_r, sel_r, sel_r, sel_r, sel_r, sel_r, g_row, w_proj)


def kernel(x, mem_kv, W_attn, W_proj, gate_bias):
    b, t, h, c = x.shape
    xf = x.reshape(t, h * c)
    mem3 = mem_kv.reshape(_M, 2, _NE)

    qkv = _qkv_call(xf, W_attn)
    y = _attn_call(qkv)
    topi = _search_call(qkv, mem_kv)
    idx = jnp.clip(topi[:, :_K], 0, _M - 1).reshape(_ROWS)
    sel = _gather_call(mem3, idx)
    sel_r = sel.reshape(_T, _K * _D)
    g_row = jnp.repeat(gate_bias.reshape(_H), _C).reshape(1, _NE)
    out = _final_call(qkv, y, sel_r, g_row, W_proj)
    return out.reshape(b, t, h * c)


# speedup vs baseline: 2.5371x; 1.0684x over previous
import functools

import jax
import jax.numpy as jnp
import numpy as np
from jax import lax
from jax.experimental import pallas as pl
from jax.experimental.pallas import tpu as pltpu
from jax.experimental.pallas import tpu_sc as plsc

_T = 2048
_H = 12
_C = 64
_NE = _H * _C
_M = 65536
_K = 3

_NEG = np.float32(-1e9)
_BIGF = np.float32(1e9)


def _qkv_body(x_ref, w_ref, o_ref):
    o_ref[...] = jnp.dot(x_ref[...], w_ref[...],
                         preferred_element_type=jnp.float32)


def _qkv_call(xf, w_attn):
    return pl.pallas_call(
        _qkv_body,
        grid=(3,),
        in_specs=[
            pl.BlockSpec((_T, _NE), lambda j: (0, 0)),
            pl.BlockSpec((_NE, _NE), lambda j: (0, j)),
        ],
        out_specs=pl.BlockSpec((_T, _NE), lambda j: (0, j)),
        out_shape=jax.ShapeDtypeStruct((_T, 3 * _NE), jnp.float32),
    )(xf, w_attn)


_QB = 512


def _attn_body(q_ref, k_ref, v_ref, o_ref):
    qb = pl.program_id(1)
    q2 = q_ref[...]
    k2 = k_ref[...]
    v2 = v_ref[...]
    rows = lax.broadcasted_iota(jnp.int32, (_QB, _T), 0) + qb * _QB
    cols = lax.broadcasted_iota(jnp.int32, (_QB, _T), 1)
    causal = cols <= rows
    halves = []
    for half in range(2):
        sl = slice(half * _C, (half + 1) * _C)
        s = lax.dot_general(q2[:, sl], k2[:, sl], (((1,), (1,)), ((), ())),
                            preferred_element_type=jnp.float32)
        s = s * np.float32(1.0 / (_C ** 0.5))
        s = jnp.where(causal, s, _NEG)
        a = jax.nn.softmax(s, axis=-1)
        halves.append(jnp.dot(a, v2[:, sl],
                              preferred_element_type=jnp.float32))
    o_ref[...] = jnp.concatenate(halves, axis=1)


def _attn_call(qkv):
    hp = _H // 2
    return pl.pallas_call(
        _attn_body,
        grid=(hp, _T // _QB),
        in_specs=[
            pl.BlockSpec((_QB, 2 * _C), lambda h2, qb: (qb, h2)),
            pl.BlockSpec((_T, 2 * _C), lambda h2, qb: (0, hp + h2)),
            pl.BlockSpec((_T, 2 * _C), lambda h2, qb: (0, 2 * hp + h2)),
        ],
        out_specs=pl.BlockSpec((_QB, 2 * _C), lambda h2, qb: (qb, h2)),
        out_shape=jax.ShapeDtypeStruct((_T, _NE), jnp.float32),
    )(qkv, qkv, qkv)


_MT = 2048


def _search_body(q_ref, mem_ref, o_ref, vals_ref, idxs_ref, mk_buf, sem):
    m = pl.program_id(0)
    nm = pl.num_programs(0)

    def key_dma(i, slot):
        return pltpu.make_async_copy(
            mem_ref.at[0, pl.ds(i * _MT, _MT), 0, :],
            mk_buf.at[slot], sem.at[slot])

    @pl.when(m == 0)
    def _():
        vals_ref[...] = jnp.full((8, _T), _NEG, jnp.float32)
        idxs_ref[...] = jnp.full((8, _T), _BIGF, jnp.float32)
        key_dma(0, 0).start()

    @pl.when(m + 1 < nm)
    def _():
        key_dma(m + 1, (m + 1) % 2).start()

    key_dma(m, m % 2).wait()
    q = q_ref[...]
    mk = mk_buf[m % 2]
    s = lax.dot_general(mk, q, (((1,), (1,)), ((), ())),
                        preferred_element_type=jnp.float32)

    rowcol = lax.broadcasted_iota(
        jnp.int32, (_MT, 1), 0).astype(jnp.float32)
    mf = (m * _MT).astype(jnp.float32)
    work = s
    bv, bi = [], []
    for j in range(_K):
        mj = jnp.max(work, axis=0, keepdims=True)
        is_m = work == mj
        aj = jnp.min(jnp.where(is_m, rowcol, _BIGF),
                     axis=0, keepdims=True) + mf
        bv.append(mj)
        bi.append(aj)
        if j < _K - 1:
            work = jnp.where(is_m, _NEG, work)

    av = [vals_ref[j:j + 1, :] for j in range(_K)]
    ai = [idxs_ref[j:j + 1, :] for j in range(_K)]
    pick = jnp.where
    c1 = bv[0] > av[0]
    r1v = pick(c1, bv[0], av[0])
    r1i = pick(c1, bi[0], ai[0])
    a2v = pick(c1, av[0], av[1])
    a2i = pick(c1, ai[0], ai[1])
    b2v = pick(c1, bv[1], bv[0])
    b2i = pick(c1, bi[1], bi[0])
    c2 = b2v > a2v
    r2v = pick(c2, b2v, a2v)
    r2i = pick(c2, b2i, a2i)
    a3v = pick(c2, a2v, pick(c1, av[1], av[2]))
    a3i = pick(c2, a2i, pick(c1, ai[1], ai[2]))
    b3v = pick(c2, pick(c1, bv[2], bv[1]), b2v)
    b3i = pick(c2, pick(c1, bi[2], bi[1]), b2i)
    c3 = b3v > a3v
    r3v = pick(c3, b3v, a3v)
    r3i = pick(c3, b3i, a3i)

    vals_ref[0:1, :] = r1v
    vals_ref[1:2, :] = r2v
    vals_ref[2:3, :] = r3v
    idxs_ref[0:1, :] = r1i
    idxs_ref[1:2, :] = r2i
    idxs_ref[2:3, :] = r3i

    @pl.when(m == nm - 1)
    def _():
        o_ref[...] = jnp.concatenate(
            [r1i, r2i, r3i, jnp.zeros((5, _T), jnp.float32)],
            axis=0).astype(jnp.int32)


def _search_call(qkv, mem_kv):
    return pl.pallas_call(
        _search_body,
        grid=(_M // _MT,),
        in_specs=[
            pl.BlockSpec((_T, _NE), lambda m: (0, 0)),
            pl.BlockSpec(memory_space=pl.ANY),
        ],
        out_specs=pl.BlockSpec((8, _T), lambda m: (0, 0)),
        out_shape=jax.ShapeDtypeStruct((8, _T), jnp.int32),
        scratch_shapes=[
            pltpu.VMEM((8, _T), jnp.float32),
            pltpu.VMEM((8, _T), jnp.float32),
            pltpu.VMEM((2, _MT, _NE), jnp.float32),
            pltpu.SemaphoreType.DMA((2,)),
        ],
    )(qkv, mem_kv)


_NW = 32
_ROWS = _T * _K
_RPW = _ROWS // _NW
_CH = 32
_D = 2 * _NE


def _sc_gather_body(table_hbm, idx_hbm, out_hbm, idx_v, rows_v, sem):
    wid = lax.axis_index("s") * 2 + lax.axis_index("c")
    base = wid * _RPW
    for chunk in range(_RPW // _CH):
        off = base + chunk * _CH
        pltpu.sync_copy(idx_hbm.at[pl.ds(off, _CH)], idx_v)
        pltpu.async_copy(table_hbm.at[idx_v], rows_v, sem).wait()
        pltpu.sync_copy(rows_v, out_hbm.at[pl.ds(off, _CH)])


@functools.cache
def _sc_gather_kernel():
    return pl.kernel(
        _sc_gather_body,
        out_type=jax.ShapeDtypeStruct((_ROWS, 2, _NE), jnp.float32),
        mesh=plsc.VectorSubcoreMesh(core_axis_name="c",
                                    subcore_axis_name="s"),
        scratch_types=[
            pltpu.VMEM((_CH,), jnp.int32),
            pltpu.VMEM((_CH, 2, _NE), jnp.float32),
            pltpu.SemaphoreType.DMA,
        ],
    )


def _gather_call(mem3, idx):
    return _sc_gather_kernel()(mem3, idx)


_SCALE = np.float32(_NE / (_H ** (-0.5)))


def _final_body(q_ref, y_ref, mk0, mk1, mk2, mv0, mv1, mv2,
                g_ref, wp_ref, o_ref):
    h2 = pl.program_id(0)
    q2 = q_ref[...]
    qk0, qk1 = [], []
    for mk in (mk0, mk1, mk2):
        p = q2 * mk[...]
        qk0.append(jnp.sum(p[:, :_C], axis=1, keepdims=True) * _SCALE)
        qk1.append(jnp.sum(p[:, _C:], axis=1, keepdims=True) * _SCALE)
    w0 = jax.nn.softmax(jnp.concatenate(qk0, axis=1), axis=1)
    w1 = jax.nn.softmax(jnp.concatenate(qk1, axis=1), axis=1)
    mem0 = jnp.zeros((_T, _C), jnp.float32)
    mem1 = jnp.zeros((_T, _C), jnp.float32)
    for k, mv in enumerate((mv0, mv1, mv2)):
        mvv = mv[...]
        mem0 += w0[:, k:k + 1] * mvv[:, :_C]
        mem1 += w1[:, k:k + 1] * mvv[:, _C:]
    mem = jnp.concatenate([mem0, mem1], axis=1)
    g = g_ref[...]
    comb = mem * g + y_ref[...] * (np.float32(1.0) - g)
    acc = jnp.dot(comb, wp_ref[...], preferred_element_type=jnp.float32)

    @pl.when(h2 == 0)
    def _():
        o_ref[...] = jnp.zeros_like(o_ref)

    o_ref[...] += acc


def _final_call(qkv, y, sel_r, g_row, w_proj):
    hp = _H // 2
    kpart = [pl.BlockSpec((_T, 2 * _C), (lambda h2, k=k: (0, k * 12 + h2)))
             for k in range(_K)]
    vpart = [pl.BlockSpec((_T, 2 * _C), (lambda h2, k=k: (0, k * 12 + hp + h2)))
             for k in range(_K)]
    return pl.pallas_call(
        _final_body,
        grid=(hp,),
        in_specs=[
            pl.BlockSpec((_T, 2 * _C), lambda h2: (0, h2)),
            pl.BlockSpec((_T, 2 * _C), lambda h2: (0, h2)),
            *kpart, *vpart,
            pl.BlockSpec((1, 2 * _C), lambda h2: (0, h2)),
            pl.BlockSpec((2 * _C, _NE), lambda h2: (h2, 0)),
        ],
        out_specs=pl.BlockSpec((_T, _NE), lambda h2: (0, 0)),
        out_shape=jax.ShapeDtypeStruct((_T, _NE), jnp.float32),
    )(qkv, y, sel_r, sel_r, sel_r, sel_r, sel_r, sel_r, g_row, w_proj)


def kernel(x, mem_kv, W_attn, W_proj, gate_bias):
    b, t, h, c = x.shape
    xf = x.reshape(t, h * c)
    mem3 = mem_kv.reshape(_M, 2, _NE)

    qkv = _qkv_call(xf, W_attn)
    y = _attn_call(qkv)
    topi = _search_call(qkv, mem_kv)
    idx = jnp.clip(topi[:_K, :].T, 0, _M - 1).reshape(_ROWS)
    sel = _gather_call(mem3, idx)
    sel_r = sel.reshape(_T, _K * _D)
    g_row = jnp.repeat(gate_bias.reshape(_H), _C).reshape(1, _NE)
    out = _final_call(qkv, y, sel_r, g_row, W_proj)
    return out.reshape(b, t, h * c)
